# baseline (speedup 1.0000x reference)
import functools

import jax
import jax.numpy as jnp
from jax import lax
from jax.experimental import pallas as pl
from jax.experimental.pallas import tpu as pltpu
from jax.experimental.pallas import tpu_sc as plsc


def _make_sc_gather(B, D, CH):
    info = plsc.get_sparse_core_info()
    NC, NS = info.num_cores, info.num_subcores
    NW = NC * NS
    assert B % NW == 0
    b_per_w = B // NW
    assert b_per_w % CH == 0
    n_chunks = b_per_w // CH

    mesh = plsc.VectorSubcoreMesh(core_axis_name="c", subcore_axis_name="s")

    @functools.partial(
        pl.kernel,
        mesh=mesh,
        out_type=jax.ShapeDtypeStruct((B, 2 * D), jnp.float32),
        scratch_types=[
            pltpu.VMEM((CH,), jnp.int32),
            pltpu.VMEM((CH,), jnp.int32),
            pltpu.VMEM((CH, D), jnp.float32),
            pltpu.VMEM((CH, D), jnp.float32),
            pltpu.SemaphoreType.DMA,
            pltpu.SemaphoreType.DMA,
            pltpu.SemaphoreType.DMA,
            pltpu.SemaphoreType.DMA,
            pltpu.SemaphoreType.DMA,
            pltpu.SemaphoreType.DMA,
        ],
        compiler_params=pltpu.CompilerParams(use_tc_tiling_on_sc=False),
    )
    def k(idx_hbm, table_hbm, out128, i0, i1, r0, r1, sg0, sg1, sw0, sw1, si0, si1):
        idx_v = (i0, i1)
        rows_v = (r0, r1)
        sg = (sg0, sg1)
        sw = (sw0, sw1)
        si = (si0, si1)
        wid = lax.axis_index("s") * NC + lax.axis_index("c")
        base = wid * b_per_w

        def out_dst(off):
            return out128.at[pl.ds(off, CH), 0:D]

        for b in range(2):
            off = base + b * CH
            pltpu.sync_copy(idx_hbm.at[pl.ds(off, CH)], idx_v[b])
            pltpu.async_copy(table_hbm.at[idx_v[b]], rows_v[b], sg[b])

        def step(i, b, prefetch):
            off = base + i * CH
            pltpu.make_async_copy(table_hbm.at[idx_v[b]], rows_v[b], sg[b]).wait()
            pltpu.async_copy(rows_v[b], out_dst(off), sw[b])
            if prefetch:
                pltpu.async_copy(
                    idx_hbm.at[pl.ds(off + 2 * CH, CH)], idx_v[b], si[b]
                )
                pltpu.make_async_copy(rows_v[b], out_dst(off), sw[b]).wait()
                pltpu.make_async_copy(
                    idx_hbm.at[pl.ds(off + 2 * CH, CH)], idx_v[b], si[b]
                ).wait()
                pltpu.async_copy(table_hbm.at[idx_v[b]], rows_v[b], sg[b])

        def outer(j, carry):
            for b in range(2):
                step(2 * j + b, b, prefetch=True)
            return carry

        lax.fori_loop(0, n_chunks // 2 - 1, outer, 0)
        for b in range(2):
            i = n_chunks - 2 + b
            step(i, b, prefetch=False)
            pltpu.make_async_copy(
                rows_v[b], out_dst(base + i * CH), sw[b]
            ).wait()

    return k


def kernel(x, table):
    Br, S = x.shape
    _, D = table.shape
    B = Br * S
    xf = x.reshape(B)
    out = _make_sc_gather(B, D, 800)(xf, table)
    return out[:, :D].reshape(Br, S, D)
